# Initial kernel scaffold; baseline (speedup 1.0000x reference)
import jax
import jax.numpy as jnp
from jax.experimental import pallas as pl


def kernel(x, edge_index, Wp, bp, convW, convB, lnG, lnB):
    raise NotImplementedError("write your pallas kernel here")


# speedup vs baseline: 2.0374x; 2.0374x over previous
import functools

import jax
import jax.numpy as jnp
from jax import lax
from jax.experimental import pallas as pl
from jax.experimental.pallas import tpu as pltpu

N = 10000
E = 160000
DIN = 256
H = 512
L = 4

NCHUNK = 4
CS = H // NCHUNK
BN = 2000
EPS = 1e-5


def _ln(r, g, b):
    mu = jnp.mean(r, axis=-1, keepdims=True)
    var = jnp.mean((r - mu) ** 2, axis=-1, keepdims=True)
    return (r - mu) * lax.rsqrt(var + EPS) * g + b


def _proj_mm_body(x_ref, Wp_ref, bp_ref, W0_ref, dinv_ref, h_ref, u_ref):
    x = x_ref[...].astype(jnp.bfloat16)
    h = jnp.dot(x, Wp_ref[...], preferred_element_type=jnp.float32) + bp_ref[...]
    h_ref[...] = h
    t = jnp.dot(h.astype(jnp.bfloat16), W0_ref[...],
                preferred_element_type=jnp.float32)
    u = t * dinv_ref[...]
    for c in range(NCHUNK):
        u_ref[c] = u[:, c * CS:(c + 1) * CS]


def _upd_body(has_next, h_ref, S_ref, u_ref, dinv_ref, cb_ref, g_ref, b_ref,
              Wn_ref, h_out, un_out):
    S = jnp.concatenate([S_ref[c] for c in range(NCHUNK)], axis=-1)
    u = jnp.concatenate([u_ref[c] for c in range(NCHUNK)], axis=-1)
    agg = dinv_ref[...] * (S + u) + cb_ref[...]
    r = h_ref[...] + jax.nn.relu(agg)
    h_new = _ln(r, g_ref[...], b_ref[...])
    h_out[...] = h_new
    if has_next:
        t = jnp.dot(h_new.astype(jnp.bfloat16), Wn_ref[...],
                    preferred_element_type=jnp.float32)
        un = t * dinv_ref[...]
        for c in range(NCHUNK):
            un_out[c] = un[:, c * CS:(c + 1) * CS]


def _row_spec():
    return pl.BlockSpec((BN, H), lambda i: (i, 0))


def _chunk_spec():
    return pl.BlockSpec((NCHUNK, BN, CS), lambda i: (0, i, 0))


def _full_spec(shape):
    return pl.BlockSpec(shape, lambda i: tuple(0 for _ in shape))


def _proj_mm(x, Wp_b, bp, W0_b, dinv2):
    grid = (N // BN,)
    return pl.pallas_call(
        _proj_mm_body,
        grid=grid,
        in_specs=[
            pl.BlockSpec((BN, DIN), lambda i: (i, 0)),
            _full_spec((DIN, H)),
            _full_spec((1, H)),
            _full_spec((H, H)),
            pl.BlockSpec((BN, 1), lambda i: (i, 0)),
        ],
        out_specs=[_row_spec(), _chunk_spec()],
        out_shape=[
            jax.ShapeDtypeStruct((N, H), jnp.float32),
            jax.ShapeDtypeStruct((NCHUNK, N, CS), jnp.float32),
        ],
    )(x, Wp_b, bp, W0_b, dinv2)


def _update(h, S, u, dinv2, cb, g, b, Wn_b):
    has_next = Wn_b is not None
    grid = (N // BN,)
    in_specs = [
        _row_spec(),
        _chunk_spec(),
        _chunk_spec(),
        pl.BlockSpec((BN, 1), lambda i: (i, 0)),
        _full_spec((1, H)),
        _full_spec((1, H)),
        _full_spec((1, H)),
    ]
    args = [h, S, u, dinv2, cb, g, b]
    out_specs = [_row_spec()]
    out_shape = [jax.ShapeDtypeStruct((N, H), jnp.float32)]
    if has_next:
        in_specs.append(_full_spec((H, H)))
        args.append(Wn_b)
        out_specs.append(_chunk_spec())
        out_shape.append(jax.ShapeDtypeStruct((NCHUNK, N, CS), jnp.float32))
    body = functools.partial(_upd_body, has_next)
    if not has_next:
        body = lambda h_ref, S_ref, u_ref, dinv_ref, cb_ref, g_ref, b_ref, h_out: (
            _upd_body(False, h_ref, S_ref, u_ref, dinv_ref, cb_ref, g_ref,
                      b_ref, None, h_out, None))
    res = pl.pallas_call(
        body, grid=grid, in_specs=in_specs, out_specs=out_specs,
        out_shape=out_shape,
    )(*args)
    return res if has_next else (res[0], None)


def _xla_scatter(u_chunk, src, dst):
    u_full = jnp.transpose(u_chunk, (1, 0, 2)).reshape(N, H)
    S = jnp.zeros((N, H), jnp.float32).at[dst].add(u_full[src])
    return jnp.transpose(S.reshape(N, NCHUNK, CS), (1, 0, 2))


def kernel(x, edge_index, Wp, bp, convW, convB, lnG, lnB):
    src = edge_index[0]
    dst = edge_index[1]
    deg = jnp.zeros((N,), jnp.float32).at[dst].add(1.0) + 1.0
    dinv = deg ** -0.5
    dinv2 = dinv[:, None]

    Wp_b = Wp.astype(jnp.bfloat16)
    convW_b = convW.astype(jnp.bfloat16)
    bp2 = bp[None, :]
    h, u = _proj_mm(x, Wp_b, bp2, convW_b[0], dinv2)
    for i in range(L):
        S = _xla_scatter(u, src, dst)
        Wn = convW_b[i + 1] if i + 1 < L else None
        h, u = _update(h, S, u, dinv2, convB[i][None, :], lnG[i][None, :],
                       lnB[i][None, :], Wn)
    return h
